# baseline (speedup 1.0000x reference)
import functools

import jax
import jax.numpy as jnp
from jax import lax
from jax.experimental import pallas as pl
from jax.experimental.pallas import tpu as pltpu
from jax.experimental.pallas import tpu_sc as plsc

VOCAB = 1000000
CTX = 200
DEMB = 64
BATCH = 4096

NC = 2
NS = 16
NW = NC * NS

TOK_PER_W = BATCH * CTX // NW
CHUNK = 128
NCHUNK = TOK_PER_W // CHUNK
NBUF = 4


def _emb_kernel(tok_hbm, e_hbm, p_hbm, out_hbm,
                idx_v, p2_v, b0, b1, b2, b3,
                sg0, sg1, sg2, sg3, ss0, ss1, ss2, ss3):
    bufs = (b0, b1, b2, b3)
    sg = (sg0, sg1, sg2, sg3)
    ss = (ss0, ss1, ss2, ss3)

    wid = lax.axis_index("s") * NC + lax.axis_index("c")
    base_row = wid * TOK_PER_W

    pltpu.sync_copy(tok_hbm.at[wid], idx_v)
    pltpu.sync_copy(p_hbm, p2_v.at[pl.ds(0, CTX)])
    pltpu.sync_copy(p_hbm, p2_v.at[pl.ds(CTX, CTX)])

    def gather_copy(jb, b):
        return pltpu.make_async_copy(e_hbm.at[idx_v.at[jb]], bufs[b], sg[b])

    def scatter_copy(jb, b):
        return pltpu.make_async_copy(
            bufs[b], out_hbm.at[pl.ds(base_row + jb * CHUNK, CHUNK)], ss[b])

    def iter_body(i, carry):
        j = i * NBUF
        for b in range(NBUF):
            jb = j + b

            @pl.when(i > 0)
            def _wait_prev_scatter():
                scatter_copy(jb - NBUF, b).wait()

            gather_copy(jb, b).start()

        for b in range(NBUF):
            jb = j + b
            gather_copy(jb, b).wait()

            pos0 = lax.rem(jb * CHUNK, CTX)
            buf = bufs[b]

            def add_row(r, carry2):
                pos = pos0 + r
                for q in range(DEMB // 16):
                    pv = p2_v[pos, pl.ds(16 * q, 16)]
                    buf[r, pl.ds(16 * q, 16)] = buf[r, pl.ds(16 * q, 16)] + pv
                return carry2

            lax.fori_loop(0, CHUNK, add_row, 0, unroll=2)

            scatter_copy(jb, b).start()
        return carry

    lax.fori_loop(0, NCHUNK // NBUF, iter_body, 0)

    for b in range(NBUF):
        scatter_copy(NCHUNK - NBUF + b, b).wait()


@functools.partial(
    pl.kernel,
    mesh=plsc.VectorSubcoreMesh(core_axis_name="c", subcore_axis_name="s"),
    out_type=jax.ShapeDtypeStruct((BATCH * CTX, DEMB), jnp.float32),
    compiler_params=pltpu.CompilerParams(use_tc_tiling_on_sc=False),
    scratch_types=[
        pltpu.VMEM((NCHUNK, CHUNK), jnp.int32),
        pltpu.VMEM((2 * CTX, DEMB), jnp.float32),
    ] + [pltpu.VMEM((CHUNK, DEMB), jnp.float32) for _ in range(NBUF)]
      + [pltpu.SemaphoreType.DMA for _ in range(2 * NBUF)],
)
def _emb_call(tok_hbm, e_hbm, p_hbm, out_hbm, *scratch):
    _emb_kernel(tok_hbm, e_hbm, p_hbm, out_hbm, *scratch)


def kernel(tokens, E, P):
    tok = tokens.astype(jnp.int32).reshape(NW, NCHUNK, CHUNK)
    out = _emb_call(tok, E, P)
    return out.reshape(BATCH, CTX, DEMB)


# speedup vs baseline: 1.2826x; 1.2826x over previous
import functools

import jax
import jax.numpy as jnp
from jax import lax
from jax.experimental import pallas as pl
from jax.experimental.pallas import tpu as pltpu
from jax.experimental.pallas import tpu_sc as plsc

VOCAB = 1000000
CTX = 200
DEMB = 64
BATCH = 4096

NC = 2
NS = 16
NW = NC * NS

TOK_PER_W = BATCH * CTX // NW
CHUNK = 128
NCHUNK = TOK_PER_W // CHUNK
NBUF = 3


def _emb_kernel(tok_hbm, e_hbm, p_hbm, out_hbm,
                idx_v, p2_v, id_v, gbufs, shared,
                sp0, sp1, sp2, sg0, sg1, sg2, sa0, sa1, sa2,
                ss0, ss1, ss2):
    sp = (sp0, sp1, sp2)
    sg = (sg0, sg1, sg2)
    sa = (sa0, sa1, sa2)
    ss = (ss0, ss1, ss2)

    sid = lax.axis_index("s")
    wid = sid * NC + lax.axis_index("c")
    base_row = wid * TOK_PER_W
    sbufs = tuple(shared.at[sid, b] for b in range(NBUF))
    bufs = tuple(gbufs.at[b] for b in range(NBUF))

    pltpu.sync_copy(tok_hbm.at[wid], idx_v)
    pltpu.sync_copy(p_hbm, p2_v.at[pl.ds(0, CTX)])
    pltpu.sync_copy(p_hbm, p2_v.at[pl.ds(CTX, CTX)])

    for q in range(CHUNK // 16):
        id_v[pl.ds(16 * q, 16)] = lax.iota(jnp.int32, 16) + 16 * q

    def prefill_copy(jb, b):
        pos0 = lax.rem(jb * CHUNK, CTX)
        return pltpu.make_async_copy(
            p2_v.at[pl.ds(pos0, CHUNK)], sbufs[b], sp[b])

    def gather_copy(jb, b):
        return pltpu.make_async_copy(e_hbm.at[idx_v.at[jb]], bufs[b], sg[b])

    def add_start(jb, b):
        pltpu.async_copy(bufs[b], sbufs[b].at[id_v], sa[b], add=True)

    def add_wait(jb, b):
        pltpu.make_async_copy(bufs[b], sbufs[b].at[id_v], sa[b]).wait()

    def scatter_copy(jb, b):
        return pltpu.make_async_copy(
            sbufs[b], out_hbm.at[pl.ds(base_row + jb * CHUNK, CHUNK)], ss[b])

    def iter_body(i, carry):
        j = i * NBUF
        for b in range(NBUF):
            jb = j + b

            @pl.when(i > 0)
            def _wait_prev_scatter():
                scatter_copy(jb - NBUF, b).wait()

            prefill_copy(jb, b).start()
            gather_copy(jb, b).start()

        for b in range(NBUF):
            jb = j + b
            prefill_copy(jb, b).wait()
            gather_copy(jb, b).wait()
            add_start(jb, b)

        for b in range(NBUF):
            jb = j + b
            add_wait(jb, b)
            scatter_copy(jb, b).start()
        return carry

    lax.fori_loop(0, NCHUNK // NBUF, iter_body, 0)

    for b in range(NBUF):
        scatter_copy(NCHUNK - NBUF + b, b).wait()


@functools.partial(
    pl.kernel,
    mesh=plsc.VectorSubcoreMesh(core_axis_name="c", subcore_axis_name="s"),
    out_type=jax.ShapeDtypeStruct((BATCH * CTX, DEMB), jnp.float32),
    compiler_params=pltpu.CompilerParams(use_tc_tiling_on_sc=False),
    scratch_types=[
        pltpu.VMEM((NCHUNK, CHUNK), jnp.int32),
        pltpu.VMEM((2 * CTX, DEMB), jnp.float32),
        pltpu.VMEM((CHUNK,), jnp.int32),
        pltpu.VMEM((NBUF, CHUNK, DEMB), jnp.float32),
        pltpu.VMEM_SHARED((NS, NBUF, CHUNK, DEMB), jnp.float32),
    ] + [pltpu.SemaphoreType.DMA for _ in range(4 * NBUF)],
)
def _emb_call(tok_hbm, e_hbm, p_hbm, out_hbm, *scratch):
    _emb_kernel(tok_hbm, e_hbm, p_hbm, out_hbm, *scratch)


def kernel(tokens, E, P):
    tok = tokens.astype(jnp.int32).reshape(NW, NCHUNK, CHUNK)
    out = _emb_call(tok, E, P)
    return out.reshape(BATCH, CTX, DEMB)


# speedup vs baseline: 1.2835x; 1.0008x over previous
import functools

import jax
import jax.numpy as jnp
from jax import lax
from jax.experimental import pallas as pl
from jax.experimental.pallas import tpu as pltpu
from jax.experimental.pallas import tpu_sc as plsc

VOCAB = 1000000
CTX = 200
DEMB = 64
BATCH = 4096

NC = 2
NS = 16
NW = NC * NS

TOK_PER_W = BATCH * CTX // NW
CHUNK = 128
NCHUNK = TOK_PER_W // CHUNK
NBUF = 3


def _emb_kernel(tok_hbm, e_hbm, p_hbm, out_hbm,
                idx_v, p2_v, id_v, gbufs, shared, sems):
    sp = tuple(sems.at[0, b] for b in range(NBUF))
    sg = tuple(sems.at[1, b] for b in range(NBUF))
    sa = tuple(sems.at[2, b] for b in range(NBUF))
    ss = tuple(sems.at[3, b] for b in range(NBUF))

    sid = lax.axis_index("s")
    wid = sid * NC + lax.axis_index("c")
    base_row = wid * TOK_PER_W
    sbufs = tuple(shared.at[sid, b] for b in range(NBUF))
    bufs = tuple(gbufs.at[b] for b in range(NBUF))

    pltpu.sync_copy(tok_hbm.at[wid], idx_v)
    pltpu.sync_copy(p_hbm, p2_v.at[pl.ds(0, CTX)])
    pltpu.sync_copy(p_hbm, p2_v.at[pl.ds(CTX, CTX)])

    for q in range(CHUNK // 16):
        id_v[pl.ds(16 * q, 16)] = lax.iota(jnp.int32, 16) + 16 * q

    def prefill_copy(jb, b):
        pos0 = lax.rem(jb * CHUNK, CTX)
        return pltpu.make_async_copy(
            p2_v.at[pl.ds(pos0, CHUNK)], sbufs[b], sp[b])

    def gather_copy(jb, b):
        return pltpu.make_async_copy(e_hbm.at[idx_v.at[jb]], bufs[b], sg[b])

    def add_start(jb, b):
        pltpu.async_copy(bufs[b], sbufs[b].at[id_v], sa[b], add=True)

    def add_wait(jb, b):
        pltpu.make_async_copy(bufs[b], sbufs[b].at[id_v], sa[b]).wait()

    def scatter_copy(jb, b):
        return pltpu.make_async_copy(
            sbufs[b], out_hbm.at[pl.ds(base_row + jb * CHUNK, CHUNK)], ss[b])

    def iter_body(i, carry):
        j = i * NBUF
        for b in range(NBUF):
            jb = j + b

            @pl.when(i > 0)
            def _wait_prev_scatter():
                scatter_copy(jb - NBUF, b).wait()

            prefill_copy(jb, b).start()
            gather_copy(jb, b).start()

        for b in range(NBUF):
            jb = j + b
            prefill_copy(jb, b).wait()
            gather_copy(jb, b).wait()
            add_start(jb, b)

        for b in range(NBUF):
            jb = j + b
            add_wait(jb, b)
            scatter_copy(jb, b).start()
        return carry

    lax.fori_loop(0, NCHUNK // NBUF, iter_body, 0)

    for b in range(NBUF):
        scatter_copy(NCHUNK - NBUF + b, b).wait()


@functools.partial(
    pl.kernel,
    mesh=plsc.VectorSubcoreMesh(core_axis_name="c", subcore_axis_name="s"),
    out_type=jax.ShapeDtypeStruct((BATCH * CTX, DEMB), jnp.float32),
    compiler_params=pltpu.CompilerParams(use_tc_tiling_on_sc=False),
    scratch_types=[
        pltpu.VMEM((NCHUNK, CHUNK), jnp.int32),
        pltpu.VMEM((2 * CTX, DEMB), jnp.float32),
        pltpu.VMEM((CHUNK,), jnp.int32),
        pltpu.VMEM((NBUF, CHUNK, DEMB), jnp.float32),
        pltpu.VMEM_SHARED((NS, NBUF, CHUNK, DEMB), jnp.float32),
        pltpu.SemaphoreType.DMA((4, NBUF)),
    ],
)
def _emb_call(tok_hbm, e_hbm, p_hbm, out_hbm, *scratch):
    _emb_kernel(tok_hbm, e_hbm, p_hbm, out_hbm, *scratch)


def kernel(tokens, E, P):
    tok = tokens.astype(jnp.int32).reshape(NW, NCHUNK, CHUNK)
    out = _emb_call(tok, E, P)
    return out.reshape(BATCH, CTX, DEMB)
